# baseline (speedup 1.0000x reference)
import functools

import jax
import jax.numpy as jnp
from jax import lax
from jax.experimental import pallas as pl
from jax.experimental.pallas import tpu as pltpu
from jax.experimental.pallas import tpu_sc as plsc

B, L, D = 128, 512, 768
N = B * L
NC, NS, LANES = 2, 16, 16
NW = NC * NS
Q = 4
DQ = D // Q
TB = NW // Q
TPB = N // TB
C = LANES
NCHUNK = TPB // C
NBUF = 4
KV = DQ // LANES


def _pack_body(tok, pos, seg, packed):
    packed[...] = tok[...] | ((seg[...] * 512 + pos[...]) << 15)


@jax.jit
def _pack(tok, pos, seg):
    return pl.pallas_call(
        _pack_body,
        out_shape=jax.ShapeDtypeStruct((B, L), jnp.int32),
    )(tok, pos, seg)


def _sc_body(packed_hbm, ttab4, loc4, out4, idx_buf, loc_loc, pbuf, sbuf, *rest):
    bufT = rest[0:NBUF]
    sem_in = rest[NBUF:2 * NBUF]
    sem_out = rest[2 * NBUF:3 * NBUF]

    wid = lax.axis_index("s") * NC + lax.axis_index("c")
    q = wid % Q
    tok_base = (wid // Q) * TPB

    pltpu.sync_copy(loc4.at[q], loc_loc)
    pltpu.sync_copy(packed_hbm.at[pl.ds(tok_base, TPB)], idx_buf)

    iota = lax.broadcasted_iota(jnp.int32, (LANES,), 0)

    def fire_in(cg, b):
        pk = idx_buf[pl.ds(cg * C, C)]
        gvec = ((pk & 0x7FFF) << 2) + q
        pltpu.async_copy(ttab4.at[gvec], bufT[b], sem_in[b])

    def drain_in(b):
        pltpu.make_async_copy(ttab4.at[pl.ds(0, C)], bufT[b], sem_in[b]).wait()

    def fire_out(cg, b):
        ovec = ((tok_base + cg * C + iota) << 2) + q
        pltpu.async_copy(bufT[b], out4.at[ovec], sem_out[b])

    def drain_out(b):
        pltpu.make_async_copy(bufT[b], out4.at[pl.ds(0, C)], sem_out[b]).wait()

    fire_in(0, 0)
    fire_in(1, 1)

    def step(qq, carry):
        for b in range(NBUF):
            cg = qq * NBUF + b
            drain_in(b)

            pk = idx_buf[pl.ds(cg * C, C)]
            pbuf[...] = (pk >> 15) & 511
            sbuf[...] = (pk >> 24) + 512

            def unit(j, carry2):
                jf = jnp.full((LANES,), j, jnp.int32)
                rp = plsc.load_gather(pbuf, [jf])
                rs = plsc.load_gather(sbuf, [jf])
                for k in range(KV):
                    col = iota + (k * LANES)
                    g1 = plsc.load_gather(loc_loc, [rp, col])
                    g2 = plsc.load_gather(loc_loc, [rs, col])
                    plsc.addupdate(bufT[b].at[j, pl.ds(k * LANES, LANES)],
                                   g1 + g2)
                return carry2

            lax.fori_loop(0, C, unit, 0)
            fire_out(cg, b)

            b2 = (b + 2) % NBUF

            @pl.when(cg >= 2)
            def _():
                drain_out(b2)

            @pl.when(cg + 2 < NCHUNK)
            def _():
                fire_in(cg + 2, b2)
        return carry

    lax.fori_loop(0, NCHUNK // NBUF, step, 0)

    drain_out((NCHUNK - 2) % NBUF)
    drain_out((NCHUNK - 1) % NBUF)


@jax.jit
def _embed_sum(packed, ttab4, loc4):
    mesh = plsc.VectorSubcoreMesh(core_axis_name="c", subcore_axis_name="s")
    scratch = [
        pltpu.VMEM((TPB,), jnp.int32),
        pltpu.VMEM((512 + 2, DQ), jnp.float32),
        pltpu.VMEM((LANES,), jnp.int32),
        pltpu.VMEM((LANES,), jnp.int32),
    ]
    scratch += [pltpu.VMEM((C, DQ), jnp.float32) for _ in range(NBUF)]
    scratch += [pltpu.SemaphoreType.DMA for _ in range(2 * NBUF)]
    f = functools.partial(
        pl.kernel,
        mesh=mesh,
        out_type=jax.ShapeDtypeStruct((N * Q, DQ), jnp.float32),
        scratch_types=scratch,
        compiler_params=pltpu.CompilerParams(
            use_tc_tiling_on_sc=False, needs_layout_passes=False),
    )(_sc_body)
    return f(packed, ttab4, loc4)


def kernel(token_ids, position_ids, segment_ids, token_table, position_table, segment_table):
    packed = _pack(token_ids.astype(jnp.int32), position_ids.astype(jnp.int32),
                   segment_ids.astype(jnp.int32))
    ttab4 = token_table.reshape(30522 * Q, DQ)
    loc4 = jnp.transpose(
        jnp.concatenate([position_table, segment_table], axis=0)
        .reshape(512 + 2, Q, DQ), (1, 0, 2))
    out = _embed_sum(packed.reshape(N), ttab4, loc4)
    return out.reshape(B, L, D)
